# baseline (speedup 1.0000x reference)
import functools

import jax
import jax.numpy as jnp
from jax import lax
from jax.experimental import pallas as pl
from jax.experimental.pallas import tpu as pltpu
from jax.experimental.pallas import tpu_sc as plsc

NC = 2
NS = 16
NW = NC * NS
CH = 128
BLK = 1024
RPT = 632
ACC_ROWS = RPT + 8
TRASH = ACC_ROWS - 1
NUM_GRAPHS = 64


def _sc_mesh():
    return plsc.VectorSubcoreMesh(
        core_axis_name="c", subcore_axis_name="s",
        num_cores=NC, num_subcores=NS)


def _make_partition_kernel(e_half, n_pad, cap):
    n_blk = e_half // BLK

    @functools.partial(
        pl.kernel,
        out_type=[
            jax.ShapeDtypeStruct((NW * cap,), jnp.int32),
            jax.ShapeDtypeStruct((NW * cap,), jnp.int32),
            jax.ShapeDtypeStruct((NW * 16,), jnp.int32),
            jax.ShapeDtypeStruct((NC * n_pad,), jnp.float32),
        ],
        mesh=_sc_mesh(),
        scratch_types=[
            pltpu.VMEM((BLK,), jnp.int32),
            pltpu.VMEM((BLK,), jnp.int32),
            pltpu.VMEM((160,), jnp.int32),
            pltpu.VMEM((160,), jnp.int32),
            pltpu.VMEM((ACC_ROWS,), jnp.float32),
        ],
        compiler_params=pltpu.CompilerParams(needs_layout_passes=False),
    )
    def part_kernel(src_hbm, dst_hbm, zeros_hbm, ssrc_hbm, sdl_hbm, cnt_hbm,
                    deg_hbm, sblk_v, dblk_v, psrc_v, pdl_v, hist_v):
        c = lax.axis_index("c")
        s = lax.axis_index("s")
        tile = c * NS + s
        lo = s * RPT
        sel_base = tile * cap
        n_sentinel = n_pad - 1
        pltpu.sync_copy(zeros_hbm.at[pl.ds(0, ACC_ROWS)], hist_v)
        ones = jnp.ones((16,), jnp.float32)

        def blk(b, carry):
            cnt, wr = carry
            base = c * e_half + b * BLK
            pltpu.sync_copy(src_hbm.at[pl.ds(base, BLK)], sblk_v)
            pltpu.sync_copy(dst_hbm.at[pl.ds(base, BLK)], dblk_v)

            def step(i, carry2):
                cnt2, wr2 = carry2
                d16 = dblk_v[pl.ds(i * 16, 16)]
                s16 = sblk_v[pl.ds(i * 16, 16)]
                dl16 = d16 - lo
                mask = (d16 >= lo) & (d16 < lo + RPT)
                dl16c = jnp.where(mask, dl16, RPT)
                plsc.addupdate_scatter(hist_v, [dl16c], ones, mask=mask)
                plsc.store_compressed(psrc_v.at[pl.ds(cnt2, 16)], s16,
                                      mask=mask)
                plsc.store_compressed(pdl_v.at[pl.ds(cnt2, 16)], dl16,
                                      mask=mask)
                cnt2 = cnt2 + jnp.sum(mask.astype(jnp.int32))

                def flush():
                    off = sel_base + wr2 * CH
                    pltpu.sync_copy(psrc_v.at[pl.ds(0, CH)],
                                    ssrc_hbm.at[pl.ds(off, CH)])
                    pltpu.sync_copy(pdl_v.at[pl.ds(0, CH)],
                                    sdl_hbm.at[pl.ds(off, CH)])
                    psrc_v[pl.ds(0, 16)] = psrc_v[pl.ds(CH, 16)]
                    pdl_v[pl.ds(0, 16)] = pdl_v[pl.ds(CH, 16)]
                do = cnt2 >= CH
                lax.cond(do, flush, lambda: None)
                return (jnp.where(do, cnt2 - CH, cnt2),
                        jnp.where(do, wr2 + 1, wr2))

            return lax.fori_loop(0, BLK // 16, step, (cnt, wr))

        cnt, wr = lax.fori_loop(0, n_blk, blk, (jnp.int32(0), jnp.int32(0)))

        def padloop(i, carry):
            keep = lax.iota(jnp.int32, 16) + i * 16 < cnt
            psrc_v[pl.ds(i * 16, 16)] = jnp.where(
                keep, psrc_v[pl.ds(i * 16, 16)],
                jnp.full((16,), n_sentinel, jnp.int32))
            pdl_v[pl.ds(i * 16, 16)] = jnp.where(
                keep, pdl_v[pl.ds(i * 16, 16)],
                jnp.full((16,), TRASH, jnp.int32))
            return carry
        lax.fori_loop(0, CH // 16, padloop, 0)

        pltpu.sync_copy(psrc_v.at[pl.ds(0, CH)],
                        ssrc_hbm.at[pl.ds(sel_base + wr * CH, CH)])
        pltpu.sync_copy(pdl_v.at[pl.ds(0, CH)],
                        sdl_hbm.at[pl.ds(sel_base + wr * CH, CH)])

        def padloop2(i, carry):
            psrc_v[pl.ds(i * 16, 16)] = jnp.full((16,), n_sentinel, jnp.int32)
            pdl_v[pl.ds(i * 16, 16)] = jnp.full((16,), TRASH, jnp.int32)
            return carry
        lax.fori_loop(0, CH // 16, padloop2, 0)
        pltpu.sync_copy(psrc_v.at[pl.ds(0, CH)],
                        ssrc_hbm.at[pl.ds(sel_base + (wr + 1) * CH, CH)])
        pltpu.sync_copy(pdl_v.at[pl.ds(0, CH)],
                        sdl_hbm.at[pl.ds(sel_base + (wr + 1) * CH, CH)])
        nch = wr + jnp.where(cnt > 0, 1, 0)
        nch = nch + (nch & 1)
        psrc_v[pl.ds(0, 16)] = jnp.full((16,), nch, jnp.int32)
        pltpu.sync_copy(psrc_v.at[pl.ds(0, 16)], cnt_hbm.at[pl.ds(tile * 16, 16)])
        pltpu.sync_copy(hist_v.at[pl.ds(0, RPT)],
                        deg_hbm.at[pl.ds(c * n_pad + lo, RPT)])

    return part_kernel


def _make_agg_kernel(n_pad, cap):
    @functools.partial(
        pl.kernel,
        out_type=jax.ShapeDtypeStruct((NC * n_pad, 64), jnp.float32),
        mesh=_sc_mesh(),
        scratch_types=[
            pltpu.VMEM((2, CH), jnp.int32),
            pltpu.VMEM((2, CH), jnp.int32),
            pltpu.VMEM((2, CH, 128), jnp.float32),
            pltpu.VMEM((16,), jnp.int32),
            pltpu.VMEM((ACC_ROWS, 64), jnp.float32),
            pltpu.SemaphoreType.DMA,
            pltpu.SemaphoreType.DMA,
        ],
        compiler_params=pltpu.CompilerParams(needs_layout_passes=False),
    )
    def agg_kernel(tab_hbm, ssrc_hbm, sdl_hbm, cnt_hbm, zeros_hbm, out_hbm,
                   src_v, dl_v, msg_v, cnt_v, acc_v, sem0, sem1):
        c = lax.axis_index("c")
        s = lax.axis_index("s")
        tile = c * NS + s
        sel_base = tile * cap
        sems = (sem0, sem1)
        pltpu.sync_copy(zeros_hbm, acc_v)
        pltpu.sync_copy(cnt_hbm.at[pl.ds(tile * 16, 16)], cnt_v)
        nch = cnt_v[pl.ds(0, 16)][0]
        last = jnp.maximum(nch - 1, 0)
        lane = lax.iota(jnp.int32, 16)

        def load_and_fire(j, q):
            off = sel_base + j * CH
            pltpu.sync_copy(ssrc_hbm.at[pl.ds(off, CH)], src_v.at[q])
            pltpu.sync_copy(sdl_hbm.at[pl.ds(off, CH)], dl_v.at[q])
            pltpu.async_copy(tab_hbm.at[src_v.at[q]], msg_v.at[q], sems[q])

        def wait_gather(q):
            pltpu.make_async_copy(tab_hbm.at[src_v.at[q]], msg_v.at[q],
                                  sems[q]).wait()

        def process(q):
            def group(gi, carry2):
                rows = gi * 16 + lane
                dls = dl_v[q, pl.ds(gi * 16, 16)]
                for t in range(64):
                    cols = (lane + t) & 63
                    vals = plsc.load_gather(msg_v.at[q], [rows, cols])
                    plsc.addupdate_scatter(acc_v, [dls, cols], vals)
                return carry2
            lax.fori_loop(0, CH // 16, group, 0)

        load_and_fire(0, 0)

        def pair(p, carry):
            i = 2 * p
            load_and_fire(jnp.minimum(i + 1, last), 1)
            wait_gather(0)
            process(0)
            load_and_fire(jnp.minimum(i + 2, last), 0)
            wait_gather(1)
            process(1)
            return carry
        lax.fori_loop(0, nch // 2, pair, 0)
        wait_gather(0)
        pltpu.sync_copy(acc_v.at[pl.ds(0, RPT)],
                        out_hbm.at[pl.ds(c * n_pad + s * RPT, RPT)])

    return agg_kernel


def _tc_prologue(x, W1, d0, d1, bn):
    n, d_in = x.shape
    d_h = W1.shape[1]

    def body(x_ref, w_ref, d0_ref, d1_ref, h_ref, g_ref, dv_ref):
        deg = d0_ref[...] + d1_ref[...] + 1.0
        dinv = lax.rsqrt(deg)
        h = jnp.dot(x_ref[...], w_ref[...], preferred_element_type=jnp.float32)
        h_ref[...] = h
        g_ref[:, 0:d_h] = h * dinv
        g_ref[:, d_h:2 * d_h] = jnp.zeros((bn, d_h), jnp.float32)
        dv_ref[...] = dinv

    return pl.pallas_call(
        body,
        grid=(n // bn,),
        in_specs=[
            pl.BlockSpec((bn, d_in), lambda i: (i, 0)),
            pl.BlockSpec((d_in, d_h), lambda i: (0, 0)),
            pl.BlockSpec((bn, 1), lambda i: (i, 0)),
            pl.BlockSpec((bn, 1), lambda i: (i, 0)),
        ],
        out_specs=[
            pl.BlockSpec((bn, d_h), lambda i: (i, 0)),
            pl.BlockSpec((bn, 2 * d_h), lambda i: (i, 0)),
            pl.BlockSpec((bn, 1), lambda i: (i, 0)),
        ],
        out_shape=[
            jax.ShapeDtypeStruct((n, d_h), jnp.float32),
            jax.ShapeDtypeStruct((n, 2 * d_h), jnp.float32),
            jax.ShapeDtypeStruct((n, 1), jnp.float32),
        ],
    )(x, W1, d0, d1)


def _tc_mid(a0, a1, h1, dinv, b1, W2, bn):
    n, d_h = h1.shape

    def body(a0_ref, a1_ref, h1_ref, dv_ref, b_ref, w_ref, h2_ref, g2_ref):
        dv = dv_ref[...]
        z = dv * (a0_ref[...] + a1_ref[...]) + (dv * dv) * h1_ref[...] + b_ref[...]
        h1p = jnp.maximum(z, 0.0)
        h2 = jnp.dot(h1p, w_ref[...], preferred_element_type=jnp.float32)
        h2_ref[...] = h2
        g2_ref[:, 0:d_h] = h2 * dv
        g2_ref[:, d_h:2 * d_h] = jnp.zeros((bn, d_h), jnp.float32)

    return pl.pallas_call(
        body,
        grid=(n // bn,),
        in_specs=[
            pl.BlockSpec((bn, d_h), lambda i: (i, 0)),
            pl.BlockSpec((bn, d_h), lambda i: (i, 0)),
            pl.BlockSpec((bn, d_h), lambda i: (i, 0)),
            pl.BlockSpec((bn, 1), lambda i: (i, 0)),
            pl.BlockSpec((1, d_h), lambda i: (0, 0)),
            pl.BlockSpec((d_h, d_h), lambda i: (0, 0)),
        ],
        out_specs=[
            pl.BlockSpec((bn, d_h), lambda i: (i, 0)),
            pl.BlockSpec((bn, 2 * d_h), lambda i: (i, 0)),
        ],
        out_shape=[
            jax.ShapeDtypeStruct((n, d_h), jnp.float32),
            jax.ShapeDtypeStruct((n, 2 * d_h), jnp.float32),
        ],
    )(a0, a1, h1, dinv, b1, W2)


def _tc_final(a0, a1, h2, dinv, b2, batch_row, Wfc, bfc):
    n, d_h = h2.shape
    ncls = Wfc.shape[1]
    g = NUM_GRAPHS

    def body(a0_ref, a1_ref, h2_ref, dv_ref, b_ref, bt_ref, wfc_ref, bfc_ref,
             o_ref):
        dv = dv_ref[...]
        z = dv * (a0_ref[...] + a1_ref[...]) + (dv * dv) * h2_ref[...] + b_ref[...]
        hp = jnp.maximum(z, 0.0)
        gids = lax.broadcasted_iota(jnp.int32, (g, n), 0)
        m_t = (bt_ref[...] == gids).astype(jnp.float32)
        sums = jnp.dot(m_t, hp, preferred_element_type=jnp.float32)
        cnt = jnp.dot(m_t, jnp.ones((n, 1), jnp.float32),
                      preferred_element_type=jnp.float32)
        pooled = sums / jnp.maximum(cnt, 1.0)
        o_ref[...] = (jnp.dot(pooled, wfc_ref[...],
                              preferred_element_type=jnp.float32)
                      + bfc_ref[...])

    return pl.pallas_call(
        body,
        out_shape=jax.ShapeDtypeStruct((g, ncls), jnp.float32),
    )(a0, a1, h2, dinv, b2, batch_row, Wfc, bfc)


def kernel(x, edge_index, batch, W1, b1, W2, b2, Wfc, bfc):
    n, d_in = x.shape
    e = edge_index.shape[1]
    d_h = W1.shape[1]

    n_pad = NS * RPT
    e_half = -(-e // (NC * BLK)) * BLK
    e_pad = NC * e_half
    cap = e_half + 2 * CH
    bn = n_pad // 4
    while bn % 8 != 0 or n_pad % bn != 0:
        bn //= 2

    epad = e_pad - e
    src = jnp.concatenate([edge_index[0], jnp.full((epad,), n, jnp.int32)])
    dst = jnp.concatenate([edge_index[1], jnp.full((epad,), n, jnp.int32)])
    x_p = jnp.concatenate([x, jnp.zeros((n_pad - n, d_in), jnp.float32)])
    batch_row = jnp.concatenate(
        [batch, jnp.full((n_pad - n,), NUM_GRAPHS, jnp.int32)]).reshape(1, n_pad)
    zeros_hist = jnp.zeros((ACC_ROWS,), jnp.float32)
    zeros_acc = jnp.zeros((ACC_ROWS, 64), jnp.float32)

    ssrc, sdl, cnts, deg = _make_partition_kernel(e_half, n_pad, cap)(
        src, dst, zeros_hist)
    deg2 = deg.reshape(NC, n_pad)
    d0 = deg2[0].reshape(n_pad, 1)
    d1 = deg2[1].reshape(n_pad, 1)

    h1, g1, dinv = _tc_prologue(x_p, W1, d0, d1, bn)

    agg_call = _make_agg_kernel(n_pad, cap)
    agg1 = agg_call(g1, ssrc, sdl, cnts, zeros_acc).reshape(NC, n_pad, d_h)
    h2, g2 = _tc_mid(agg1[0], agg1[1], h1, dinv, b1.reshape(1, -1), W2, bn)
    agg2 = agg_call(g2, ssrc, sdl, cnts, zeros_acc).reshape(NC, n_pad, d_h)

    return _tc_final(agg2[0], agg2[1], h2, dinv, b2.reshape(1, -1), batch_row,
                     Wfc, bfc.reshape(1, -1))


# speedup vs baseline: 1.0413x; 1.0413x over previous
import functools

import jax
import jax.numpy as jnp
from jax import lax
from jax.experimental import pallas as pl
from jax.experimental.pallas import tpu as pltpu
from jax.experimental.pallas import tpu_sc as plsc

NC = 2
NS = 16
NW = NC * NS
CH = 128
BLK = 1024
RPT = 632
ACC_ROWS = RPT + 8
TRASH = ACC_ROWS - 1
NUM_GRAPHS = 64


def _sc_mesh():
    return plsc.VectorSubcoreMesh(
        core_axis_name="c", subcore_axis_name="s",
        num_cores=NC, num_subcores=NS)


def _make_partition_kernel(e_half, n_pad, cap):
    n_blk = e_half // BLK

    @functools.partial(
        pl.kernel,
        out_type=[
            jax.ShapeDtypeStruct((NW * cap,), jnp.int32),
            jax.ShapeDtypeStruct((NW * cap,), jnp.int32),
            jax.ShapeDtypeStruct((NW * 16,), jnp.int32),
            jax.ShapeDtypeStruct((NC * n_pad,), jnp.float32),
        ],
        mesh=_sc_mesh(),
        scratch_types=[
            pltpu.VMEM((BLK,), jnp.int32),
            pltpu.VMEM((BLK,), jnp.int32),
            pltpu.VMEM((160,), jnp.int32),
            pltpu.VMEM((160,), jnp.int32),
            pltpu.VMEM((ACC_ROWS,), jnp.float32),
        ],
        compiler_params=pltpu.CompilerParams(needs_layout_passes=False),
    )
    def part_kernel(src_hbm, dst_hbm, zeros_hbm, ssrc_hbm, sdl_hbm, cnt_hbm,
                    deg_hbm, sblk_v, dblk_v, psrc_v, pdl_v, hist_v):
        c = lax.axis_index("c")
        s = lax.axis_index("s")
        tile = c * NS + s
        lo = s * RPT
        sel_base = tile * cap
        n_sentinel = n_pad - 1
        pltpu.sync_copy(zeros_hbm.at[pl.ds(0, ACC_ROWS)], hist_v)
        ones = jnp.ones((16,), jnp.float32)

        def blk(b, carry):
            cnt, wr = carry
            base = c * e_half + b * BLK
            pltpu.sync_copy(src_hbm.at[pl.ds(base, BLK)], sblk_v)
            pltpu.sync_copy(dst_hbm.at[pl.ds(base, BLK)], dblk_v)

            def step(i, carry2):
                cnt2, wr2 = carry2
                d16 = dblk_v[pl.ds(i * 16, 16)]
                s16 = sblk_v[pl.ds(i * 16, 16)]
                dl16 = d16 - lo
                mask = (d16 >= lo) & (d16 < lo + RPT)
                dl16c = jnp.where(mask, dl16, RPT)
                plsc.addupdate_scatter(hist_v, [dl16c], ones, mask=mask)
                plsc.store_compressed(psrc_v.at[pl.ds(cnt2, 16)], s16,
                                      mask=mask)
                plsc.store_compressed(pdl_v.at[pl.ds(cnt2, 16)], dl16,
                                      mask=mask)
                cnt2 = cnt2 + jnp.sum(mask.astype(jnp.int32))

                def flush():
                    off = sel_base + wr2 * CH
                    pltpu.sync_copy(psrc_v.at[pl.ds(0, CH)],
                                    ssrc_hbm.at[pl.ds(off, CH)])
                    pltpu.sync_copy(pdl_v.at[pl.ds(0, CH)],
                                    sdl_hbm.at[pl.ds(off, CH)])
                    psrc_v[pl.ds(0, 16)] = psrc_v[pl.ds(CH, 16)]
                    pdl_v[pl.ds(0, 16)] = pdl_v[pl.ds(CH, 16)]
                do = cnt2 >= CH
                lax.cond(do, flush, lambda: None)
                return (jnp.where(do, cnt2 - CH, cnt2),
                        jnp.where(do, wr2 + 1, wr2))

            return lax.fori_loop(0, BLK // 16, step, (cnt, wr))

        cnt, wr = lax.fori_loop(0, n_blk, blk, (jnp.int32(0), jnp.int32(0)))

        def padloop(i, carry):
            keep = lax.iota(jnp.int32, 16) + i * 16 < cnt
            psrc_v[pl.ds(i * 16, 16)] = jnp.where(
                keep, psrc_v[pl.ds(i * 16, 16)],
                jnp.full((16,), n_sentinel, jnp.int32))
            pdl_v[pl.ds(i * 16, 16)] = jnp.where(
                keep, pdl_v[pl.ds(i * 16, 16)],
                jnp.full((16,), TRASH, jnp.int32))
            return carry
        lax.fori_loop(0, CH // 16, padloop, 0)

        pltpu.sync_copy(psrc_v.at[pl.ds(0, CH)],
                        ssrc_hbm.at[pl.ds(sel_base + wr * CH, CH)])
        pltpu.sync_copy(pdl_v.at[pl.ds(0, CH)],
                        sdl_hbm.at[pl.ds(sel_base + wr * CH, CH)])

        def padloop2(i, carry):
            psrc_v[pl.ds(i * 16, 16)] = jnp.full((16,), n_sentinel, jnp.int32)
            pdl_v[pl.ds(i * 16, 16)] = jnp.full((16,), TRASH, jnp.int32)
            return carry
        lax.fori_loop(0, CH // 16, padloop2, 0)
        pltpu.sync_copy(psrc_v.at[pl.ds(0, CH)],
                        ssrc_hbm.at[pl.ds(sel_base + (wr + 1) * CH, CH)])
        pltpu.sync_copy(pdl_v.at[pl.ds(0, CH)],
                        sdl_hbm.at[pl.ds(sel_base + (wr + 1) * CH, CH)])
        nch = wr + jnp.where(cnt > 0, 1, 0)
        nch = nch + (nch & 1)
        psrc_v[pl.ds(0, 16)] = jnp.full((16,), nch, jnp.int32)
        pltpu.sync_copy(psrc_v.at[pl.ds(0, 16)], cnt_hbm.at[pl.ds(tile * 16, 16)])
        pltpu.sync_copy(hist_v.at[pl.ds(0, RPT)],
                        deg_hbm.at[pl.ds(c * n_pad + lo, RPT)])

    return part_kernel


def _make_agg_kernel(n_pad, cap):
    @functools.partial(
        pl.kernel,
        out_type=jax.ShapeDtypeStruct((NC * n_pad, 64), jnp.float32),
        mesh=_sc_mesh(),
        scratch_types=[
            pltpu.VMEM((CH,), jnp.int32),
            pltpu.VMEM((CH,), jnp.int32),
            pltpu.VMEM((CH, 128), jnp.float32),
            pltpu.VMEM((16,), jnp.int32),
            pltpu.VMEM((ACC_ROWS, 64), jnp.float32),
            pltpu.SemaphoreType.DMA,
        ],
        compiler_params=pltpu.CompilerParams(needs_layout_passes=False),
    )
    def agg_kernel(tab_hbm, ssrc_hbm, sdl_hbm, cnt_hbm, zeros_hbm, out_hbm,
                   src_v, dl_v, msg_v, cnt_v, acc_v, sem):
        c = lax.axis_index("c")
        s = lax.axis_index("s")
        tile = c * NS + s
        sel_base = tile * cap
        pltpu.sync_copy(zeros_hbm, acc_v)
        pltpu.sync_copy(cnt_hbm.at[pl.ds(tile * 16, 16)], cnt_v)
        nch = cnt_v[pl.ds(0, 16)][0]
        lane = lax.iota(jnp.int32, 16)

        def chunk(i, carry):
            off = sel_base + i * CH
            pltpu.sync_copy(ssrc_hbm.at[pl.ds(off, CH)], src_v)
            pltpu.sync_copy(sdl_hbm.at[pl.ds(off, CH)], dl_v)
            pltpu.async_copy(tab_hbm.at[src_v], msg_v, sem).wait()

            def group(gi, carry2):
                rows = gi * 16 + lane
                dls = dl_v[pl.ds(gi * 16, 16)]
                for t in range(64):
                    cols = (lane + t) & 63
                    vals = plsc.load_gather(msg_v, [rows, cols])
                    plsc.addupdate_scatter(acc_v, [dls, cols], vals)
                return carry2
            lax.fori_loop(0, CH // 16, group, 0)
            return carry
        lax.fori_loop(0, nch, chunk, 0)
        pltpu.sync_copy(acc_v.at[pl.ds(0, RPT)],
                        out_hbm.at[pl.ds(c * n_pad + s * RPT, RPT)])

    return agg_kernel


def _tc_prologue(x, W1, d0, d1, bn):
    n, d_in = x.shape
    d_h = W1.shape[1]

    def body(x_ref, w_ref, d0_ref, d1_ref, h_ref, g_ref, dv_ref):
        deg = d0_ref[...] + d1_ref[...] + 1.0
        dinv = lax.rsqrt(deg)
        h = jnp.dot(x_ref[...], w_ref[...], preferred_element_type=jnp.float32)
        h_ref[...] = h
        g_ref[:, 0:d_h] = h * dinv
        g_ref[:, d_h:2 * d_h] = jnp.zeros((bn, d_h), jnp.float32)
        dv_ref[...] = dinv

    return pl.pallas_call(
        body,
        grid=(n // bn,),
        in_specs=[
            pl.BlockSpec((bn, d_in), lambda i: (i, 0)),
            pl.BlockSpec((d_in, d_h), lambda i: (0, 0)),
            pl.BlockSpec((bn, 1), lambda i: (i, 0)),
            pl.BlockSpec((bn, 1), lambda i: (i, 0)),
        ],
        out_specs=[
            pl.BlockSpec((bn, d_h), lambda i: (i, 0)),
            pl.BlockSpec((bn, 2 * d_h), lambda i: (i, 0)),
            pl.BlockSpec((bn, 1), lambda i: (i, 0)),
        ],
        out_shape=[
            jax.ShapeDtypeStruct((n, d_h), jnp.float32),
            jax.ShapeDtypeStruct((n, 2 * d_h), jnp.float32),
            jax.ShapeDtypeStruct((n, 1), jnp.float32),
        ],
    )(x, W1, d0, d1)


def _tc_mid(a0, a1, h1, dinv, b1, W2, bn):
    n, d_h = h1.shape

    def body(a0_ref, a1_ref, h1_ref, dv_ref, b_ref, w_ref, h2_ref, g2_ref):
        dv = dv_ref[...]
        z = dv * (a0_ref[...] + a1_ref[...]) + (dv * dv) * h1_ref[...] + b_ref[...]
        h1p = jnp.maximum(z, 0.0)
        h2 = jnp.dot(h1p, w_ref[...], preferred_element_type=jnp.float32)
        h2_ref[...] = h2
        g2_ref[:, 0:d_h] = h2 * dv
        g2_ref[:, d_h:2 * d_h] = jnp.zeros((bn, d_h), jnp.float32)

    return pl.pallas_call(
        body,
        grid=(n // bn,),
        in_specs=[
            pl.BlockSpec((bn, d_h), lambda i: (i, 0)),
            pl.BlockSpec((bn, d_h), lambda i: (i, 0)),
            pl.BlockSpec((bn, d_h), lambda i: (i, 0)),
            pl.BlockSpec((bn, 1), lambda i: (i, 0)),
            pl.BlockSpec((1, d_h), lambda i: (0, 0)),
            pl.BlockSpec((d_h, d_h), lambda i: (0, 0)),
        ],
        out_specs=[
            pl.BlockSpec((bn, d_h), lambda i: (i, 0)),
            pl.BlockSpec((bn, 2 * d_h), lambda i: (i, 0)),
        ],
        out_shape=[
            jax.ShapeDtypeStruct((n, d_h), jnp.float32),
            jax.ShapeDtypeStruct((n, 2 * d_h), jnp.float32),
        ],
    )(a0, a1, h1, dinv, b1, W2)


def _tc_final(a0, a1, h2, dinv, b2, batch_row, Wfc, bfc):
    n, d_h = h2.shape
    ncls = Wfc.shape[1]
    g = NUM_GRAPHS

    def body(a0_ref, a1_ref, h2_ref, dv_ref, b_ref, bt_ref, wfc_ref, bfc_ref,
             o_ref):
        dv = dv_ref[...]
        z = dv * (a0_ref[...] + a1_ref[...]) + (dv * dv) * h2_ref[...] + b_ref[...]
        hp = jnp.maximum(z, 0.0)
        gids = lax.broadcasted_iota(jnp.int32, (g, n), 0)
        m_t = (bt_ref[...] == gids).astype(jnp.float32)
        sums = jnp.dot(m_t, hp, preferred_element_type=jnp.float32)
        cnt = jnp.dot(m_t, jnp.ones((n, 1), jnp.float32),
                      preferred_element_type=jnp.float32)
        pooled = sums / jnp.maximum(cnt, 1.0)
        o_ref[...] = (jnp.dot(pooled, wfc_ref[...],
                              preferred_element_type=jnp.float32)
                      + bfc_ref[...])

    return pl.pallas_call(
        body,
        out_shape=jax.ShapeDtypeStruct((g, ncls), jnp.float32),
    )(a0, a1, h2, dinv, b2, batch_row, Wfc, bfc)


def kernel(x, edge_index, batch, W1, b1, W2, b2, Wfc, bfc):
    n, d_in = x.shape
    e = edge_index.shape[1]
    d_h = W1.shape[1]

    n_pad = NS * RPT
    e_half = -(-e // (NC * BLK)) * BLK
    e_pad = NC * e_half
    cap = e_half + 2 * CH
    bn = n_pad // 4
    while bn % 8 != 0 or n_pad % bn != 0:
        bn //= 2

    epad = e_pad - e
    src = jnp.concatenate([edge_index[0], jnp.full((epad,), n, jnp.int32)])
    dst = jnp.concatenate([edge_index[1], jnp.full((epad,), n, jnp.int32)])
    x_p = jnp.concatenate([x, jnp.zeros((n_pad - n, d_in), jnp.float32)])
    batch_row = jnp.concatenate(
        [batch, jnp.full((n_pad - n,), NUM_GRAPHS, jnp.int32)]).reshape(1, n_pad)
    zeros_hist = jnp.zeros((ACC_ROWS,), jnp.float32)
    zeros_acc = jnp.zeros((ACC_ROWS, 64), jnp.float32)

    ssrc, sdl, cnts, deg = _make_partition_kernel(e_half, n_pad, cap)(
        src, dst, zeros_hist)
    deg2 = deg.reshape(NC, n_pad)
    d0 = deg2[0].reshape(n_pad, 1)
    d1 = deg2[1].reshape(n_pad, 1)

    h1, g1, dinv = _tc_prologue(x_p, W1, d0, d1, bn)

    agg_call = _make_agg_kernel(n_pad, cap)
    agg1 = agg_call(g1, ssrc, sdl, cnts, zeros_acc).reshape(NC, n_pad, d_h)
    h2, g2 = _tc_mid(agg1[0], agg1[1], h1, dinv, b1.reshape(1, -1), W2, bn)
    agg2 = agg_call(g2, ssrc, sdl, cnts, zeros_acc).reshape(NC, n_pad, d_h)

    return _tc_final(agg2[0], agg2[1], h2, dinv, b2.reshape(1, -1), batch_row,
                     Wfc, bfc.reshape(1, -1))


# speedup vs baseline: 1.2201x; 1.1718x over previous
import functools

import jax
import jax.numpy as jnp
from jax import lax
from jax.experimental import pallas as pl
from jax.experimental.pallas import tpu as pltpu
from jax.experimental.pallas import tpu_sc as plsc

NC = 2
NS = 16
NW = NC * NS
CH = 128
BLK = 1024
RPT = 632
ACC_ROWS = RPT + 8
TRASH = ACC_ROWS - 1
NUM_GRAPHS = 64


def _sc_mesh():
    return plsc.VectorSubcoreMesh(
        core_axis_name="c", subcore_axis_name="s",
        num_cores=NC, num_subcores=NS)


def _make_partition_kernel(e_half, n_pad, cap):
    n_blk = e_half // BLK

    @functools.partial(
        pl.kernel,
        out_type=[
            jax.ShapeDtypeStruct((NW * cap,), jnp.int32),
            jax.ShapeDtypeStruct((NW * cap,), jnp.int32),
            jax.ShapeDtypeStruct((NW * 16,), jnp.int32),
            jax.ShapeDtypeStruct((NC * n_pad,), jnp.float32),
        ],
        mesh=_sc_mesh(),
        scratch_types=[
            pltpu.VMEM((BLK,), jnp.int32),
            pltpu.VMEM((BLK,), jnp.int32),
            pltpu.VMEM((160,), jnp.int32),
            pltpu.VMEM((160,), jnp.int32),
            pltpu.VMEM((ACC_ROWS,), jnp.float32),
        ],
        compiler_params=pltpu.CompilerParams(needs_layout_passes=False),
    )
    def part_kernel(src_hbm, dst_hbm, zeros_hbm, ssrc_hbm, sdl_hbm, cnt_hbm,
                    deg_hbm, sblk_v, dblk_v, psrc_v, pdl_v, hist_v):
        c = lax.axis_index("c")
        s = lax.axis_index("s")
        tile = c * NS + s
        lo = s * RPT
        sel_base = tile * cap
        n_sentinel = n_pad - 1
        pltpu.sync_copy(zeros_hbm.at[pl.ds(0, ACC_ROWS)], hist_v)
        ones = jnp.ones((16,), jnp.float32)

        def blk(b, carry):
            cnt, wr = carry
            base = c * e_half + b * BLK
            pltpu.sync_copy(src_hbm.at[pl.ds(base, BLK)], sblk_v)
            pltpu.sync_copy(dst_hbm.at[pl.ds(base, BLK)], dblk_v)

            def step(i, carry2):
                cnt2, wr2 = carry2
                d16 = dblk_v[pl.ds(i * 16, 16)]
                s16 = sblk_v[pl.ds(i * 16, 16)]
                dl16 = d16 - lo
                mask = (d16 >= lo) & (d16 < lo + RPT)
                dl16c = jnp.where(mask, dl16, RPT)
                plsc.addupdate_scatter(hist_v, [dl16c], ones, mask=mask)
                plsc.store_compressed(psrc_v.at[pl.ds(cnt2, 16)], s16,
                                      mask=mask)
                plsc.store_compressed(pdl_v.at[pl.ds(cnt2, 16)], dl16,
                                      mask=mask)
                cnt2 = cnt2 + jnp.sum(mask.astype(jnp.int32))

                def flush():
                    off = sel_base + wr2 * CH
                    pltpu.sync_copy(psrc_v.at[pl.ds(0, CH)],
                                    ssrc_hbm.at[pl.ds(off, CH)])
                    pltpu.sync_copy(pdl_v.at[pl.ds(0, CH)],
                                    sdl_hbm.at[pl.ds(off, CH)])
                    psrc_v[pl.ds(0, 16)] = psrc_v[pl.ds(CH, 16)]
                    pdl_v[pl.ds(0, 16)] = pdl_v[pl.ds(CH, 16)]
                do = cnt2 >= CH
                lax.cond(do, flush, lambda: None)
                return (jnp.where(do, cnt2 - CH, cnt2),
                        jnp.where(do, wr2 + 1, wr2))

            return lax.fori_loop(0, BLK // 16, step, (cnt, wr))

        cnt, wr = lax.fori_loop(0, n_blk, blk, (jnp.int32(0), jnp.int32(0)))

        spread = (lax.iota(jnp.int32, 16) * 577 + s * 37) & 8191

        def padloop(i, carry):
            keep = lax.iota(jnp.int32, 16) + i * 16 < cnt
            psrc_v[pl.ds(i * 16, 16)] = jnp.where(
                keep, psrc_v[pl.ds(i * 16, 16)], spread + i)
            pdl_v[pl.ds(i * 16, 16)] = jnp.where(
                keep, pdl_v[pl.ds(i * 16, 16)],
                jnp.full((16,), TRASH, jnp.int32))
            return carry
        lax.fori_loop(0, CH // 16, padloop, 0)

        pltpu.sync_copy(psrc_v.at[pl.ds(0, CH)],
                        ssrc_hbm.at[pl.ds(sel_base + wr * CH, CH)])
        pltpu.sync_copy(pdl_v.at[pl.ds(0, CH)],
                        sdl_hbm.at[pl.ds(sel_base + wr * CH, CH)])

        def padloop2(i, carry):
            psrc_v[pl.ds(i * 16, 16)] = spread + i * 3
            pdl_v[pl.ds(i * 16, 16)] = jnp.full((16,), TRASH, jnp.int32)
            return carry
        lax.fori_loop(0, CH // 16, padloop2, 0)
        pltpu.sync_copy(psrc_v.at[pl.ds(0, CH)],
                        ssrc_hbm.at[pl.ds(sel_base + (wr + 1) * CH, CH)])
        pltpu.sync_copy(pdl_v.at[pl.ds(0, CH)],
                        sdl_hbm.at[pl.ds(sel_base + (wr + 1) * CH, CH)])
        nch = wr + jnp.where(cnt > 0, 1, 0)
        nch = nch + (nch & 1)
        psrc_v[pl.ds(0, 16)] = jnp.full((16,), nch, jnp.int32)
        pltpu.sync_copy(psrc_v.at[pl.ds(0, 16)], cnt_hbm.at[pl.ds(tile * 16, 16)])
        pltpu.sync_copy(hist_v.at[pl.ds(0, RPT)],
                        deg_hbm.at[pl.ds(c * n_pad + lo, RPT)])

    return part_kernel


def _make_agg_kernel(n_pad, cap):
    @functools.partial(
        pl.kernel,
        out_type=jax.ShapeDtypeStruct((NC * n_pad, 64), jnp.float32),
        mesh=_sc_mesh(),
        scratch_types=[
            pltpu.VMEM((CH,), jnp.int32),
            pltpu.VMEM((CH,), jnp.int32),
            pltpu.VMEM((CH, 128), jnp.float32),
            pltpu.VMEM((16,), jnp.int32),
            pltpu.VMEM((ACC_ROWS, 64), jnp.float32),
            pltpu.SemaphoreType.DMA,
        ],
        compiler_params=pltpu.CompilerParams(needs_layout_passes=False),
    )
    def agg_kernel(tab_hbm, ssrc_hbm, sdl_hbm, cnt_hbm, zeros_hbm, out_hbm,
                   src_v, dl_v, msg_v, cnt_v, acc_v, sem):
        c = lax.axis_index("c")
        s = lax.axis_index("s")
        tile = c * NS + s
        sel_base = tile * cap
        pltpu.sync_copy(zeros_hbm, acc_v)
        pltpu.sync_copy(cnt_hbm.at[pl.ds(tile * 16, 16)], cnt_v)
        nch = cnt_v[pl.ds(0, 16)][0]
        lane = lax.iota(jnp.int32, 16)

        def chunk(i, carry):
            off = sel_base + i * CH
            pltpu.sync_copy(ssrc_hbm.at[pl.ds(off, CH)], src_v)
            pltpu.sync_copy(sdl_hbm.at[pl.ds(off, CH)], dl_v)
            pltpu.async_copy(tab_hbm.at[src_v], msg_v, sem).wait()

            def group(gi, carry2):
                rows = gi * 16 + lane
                dls = dl_v[pl.ds(gi * 16, 16)]
                for t in range(64):
                    cols = (lane + t) & 63
                    vals = plsc.load_gather(msg_v, [rows, cols])
                    plsc.addupdate_scatter(acc_v, [dls, cols], vals)
                return carry2
            lax.fori_loop(0, CH // 16, group, 0)
            return carry
        lax.fori_loop(0, nch, chunk, 0)
        pltpu.sync_copy(acc_v.at[pl.ds(0, RPT)],
                        out_hbm.at[pl.ds(c * n_pad + s * RPT, RPT)])

    return agg_kernel


def _tc_prologue(x, W1, d0, d1, bn):
    n, d_in = x.shape
    d_h = W1.shape[1]

    def body(x_ref, w_ref, d0_ref, d1_ref, h_ref, g_ref, dv_ref):
        deg = d0_ref[...] + d1_ref[...] + 1.0
        dinv = lax.rsqrt(deg)
        h = jnp.dot(x_ref[...], w_ref[...], preferred_element_type=jnp.float32)
        h_ref[...] = h
        g_ref[:, 0:d_h] = h * dinv
        g_ref[:, d_h:2 * d_h] = jnp.zeros((bn, d_h), jnp.float32)
        dv_ref[...] = dinv

    return pl.pallas_call(
        body,
        grid=(n // bn,),
        in_specs=[
            pl.BlockSpec((bn, d_in), lambda i: (i, 0)),
            pl.BlockSpec((d_in, d_h), lambda i: (0, 0)),
            pl.BlockSpec((bn, 1), lambda i: (i, 0)),
            pl.BlockSpec((bn, 1), lambda i: (i, 0)),
        ],
        out_specs=[
            pl.BlockSpec((bn, d_h), lambda i: (i, 0)),
            pl.BlockSpec((bn, 2 * d_h), lambda i: (i, 0)),
            pl.BlockSpec((bn, 1), lambda i: (i, 0)),
        ],
        out_shape=[
            jax.ShapeDtypeStruct((n, d_h), jnp.float32),
            jax.ShapeDtypeStruct((n, 2 * d_h), jnp.float32),
            jax.ShapeDtypeStruct((n, 1), jnp.float32),
        ],
    )(x, W1, d0, d1)


def _tc_mid(a0, a1, h1, dinv, b1, W2, bn):
    n, d_h = h1.shape

    def body(a0_ref, a1_ref, h1_ref, dv_ref, b_ref, w_ref, h2_ref, g2_ref):
        dv = dv_ref[...]
        z = dv * (a0_ref[...] + a1_ref[...]) + (dv * dv) * h1_ref[...] + b_ref[...]
        h1p = jnp.maximum(z, 0.0)
        h2 = jnp.dot(h1p, w_ref[...], preferred_element_type=jnp.float32)
        h2_ref[...] = h2
        g2_ref[:, 0:d_h] = h2 * dv
        g2_ref[:, d_h:2 * d_h] = jnp.zeros((bn, d_h), jnp.float32)

    return pl.pallas_call(
        body,
        grid=(n // bn,),
        in_specs=[
            pl.BlockSpec((bn, d_h), lambda i: (i, 0)),
            pl.BlockSpec((bn, d_h), lambda i: (i, 0)),
            pl.BlockSpec((bn, d_h), lambda i: (i, 0)),
            pl.BlockSpec((bn, 1), lambda i: (i, 0)),
            pl.BlockSpec((1, d_h), lambda i: (0, 0)),
            pl.BlockSpec((d_h, d_h), lambda i: (0, 0)),
        ],
        out_specs=[
            pl.BlockSpec((bn, d_h), lambda i: (i, 0)),
            pl.BlockSpec((bn, 2 * d_h), lambda i: (i, 0)),
        ],
        out_shape=[
            jax.ShapeDtypeStruct((n, d_h), jnp.float32),
            jax.ShapeDtypeStruct((n, 2 * d_h), jnp.float32),
        ],
    )(a0, a1, h1, dinv, b1, W2)


def _tc_final(a0, a1, h2, dinv, b2, batch_row, Wfc, bfc):
    n, d_h = h2.shape
    ncls = Wfc.shape[1]
    g = NUM_GRAPHS

    def body(a0_ref, a1_ref, h2_ref, dv_ref, b_ref, bt_ref, wfc_ref, bfc_ref,
             o_ref):
        dv = dv_ref[...]
        z = dv * (a0_ref[...] + a1_ref[...]) + (dv * dv) * h2_ref[...] + b_ref[...]
        hp = jnp.maximum(z, 0.0)
        gids = lax.broadcasted_iota(jnp.int32, (g, n), 0)
        m_t = (bt_ref[...] == gids).astype(jnp.float32)
        sums = jnp.dot(m_t, hp, preferred_element_type=jnp.float32)
        cnt = jnp.dot(m_t, jnp.ones((n, 1), jnp.float32),
                      preferred_element_type=jnp.float32)
        pooled = sums / jnp.maximum(cnt, 1.0)
        o_ref[...] = (jnp.dot(pooled, wfc_ref[...],
                              preferred_element_type=jnp.float32)
                      + bfc_ref[...])

    return pl.pallas_call(
        body,
        out_shape=jax.ShapeDtypeStruct((g, ncls), jnp.float32),
    )(a0, a1, h2, dinv, b2, batch_row, Wfc, bfc)


def kernel(x, edge_index, batch, W1, b1, W2, b2, Wfc, bfc):
    n, d_in = x.shape
    e = edge_index.shape[1]
    d_h = W1.shape[1]

    n_pad = NS * RPT
    e_half = -(-e // (NC * BLK)) * BLK
    e_pad = NC * e_half
    cap = e_half + 2 * CH
    bn = n_pad // 4
    while bn % 8 != 0 or n_pad % bn != 0:
        bn //= 2

    epad = e_pad - e
    src = jnp.concatenate([edge_index[0], jnp.full((epad,), n, jnp.int32)])
    dst = jnp.concatenate([edge_index[1], jnp.full((epad,), n, jnp.int32)])
    x_p = jnp.concatenate([x, jnp.zeros((n_pad - n, d_in), jnp.float32)])
    batch_row = jnp.concatenate(
        [batch, jnp.full((n_pad - n,), NUM_GRAPHS, jnp.int32)]).reshape(1, n_pad)
    zeros_hist = jnp.zeros((ACC_ROWS,), jnp.float32)
    zeros_acc = jnp.zeros((ACC_ROWS, 64), jnp.float32)

    ssrc, sdl, cnts, deg = _make_partition_kernel(e_half, n_pad, cap)(
        src, dst, zeros_hist)
    deg2 = deg.reshape(NC, n_pad)
    d0 = deg2[0].reshape(n_pad, 1)
    d1 = deg2[1].reshape(n_pad, 1)

    h1, g1, dinv = _tc_prologue(x_p, W1, d0, d1, bn)

    agg_call = _make_agg_kernel(n_pad, cap)
    agg1 = agg_call(g1, ssrc, sdl, cnts, zeros_acc).reshape(NC, n_pad, d_h)
    h2, g2 = _tc_mid(agg1[0], agg1[1], h1, dinv, b1.reshape(1, -1), W2, bn)
    agg2 = agg_call(g2, ssrc, sdl, cnts, zeros_acc).reshape(NC, n_pad, d_h)

    return _tc_final(agg2[0], agg2[1], h2, dinv, b2.reshape(1, -1), batch_row,
                     Wfc, bfc.reshape(1, -1))


# speedup vs baseline: 1.3012x; 1.0664x over previous
import functools

import jax
import jax.numpy as jnp
from jax import lax
from jax.experimental import pallas as pl
from jax.experimental.pallas import tpu as pltpu
from jax.experimental.pallas import tpu_sc as plsc

NC = 2
NS = 16
NW = NC * NS
CH = 128
BLK = 1024
RPT = 632
ACC_ROWS = RPT + 8
TRASH = ACC_ROWS - 1
NUM_GRAPHS = 64


def _sc_mesh():
    return plsc.VectorSubcoreMesh(
        core_axis_name="c", subcore_axis_name="s",
        num_cores=NC, num_subcores=NS)


def _make_partition_kernel(e_half, n_pad, cap):
    n_blk = e_half // BLK

    @functools.partial(
        pl.kernel,
        out_type=[
            jax.ShapeDtypeStruct((NW * cap,), jnp.int32),
            jax.ShapeDtypeStruct((NW * cap,), jnp.int32),
            jax.ShapeDtypeStruct((NW * 16,), jnp.int32),
            jax.ShapeDtypeStruct((NC * n_pad,), jnp.float32),
        ],
        mesh=_sc_mesh(),
        scratch_types=[
            pltpu.VMEM((BLK,), jnp.int32),
            pltpu.VMEM((BLK,), jnp.int32),
            pltpu.VMEM((160,), jnp.int32),
            pltpu.VMEM((160,), jnp.int32),
            pltpu.VMEM((ACC_ROWS,), jnp.float32),
        ],
        compiler_params=pltpu.CompilerParams(needs_layout_passes=False),
    )
    def part_kernel(src_hbm, dst_hbm, zeros_hbm, ssrc_hbm, sdl_hbm, cnt_hbm,
                    deg_hbm, sblk_v, dblk_v, psrc_v, pdl_v, hist_v):
        c = lax.axis_index("c")
        s = lax.axis_index("s")
        tile = c * NS + s
        lo = s * RPT
        sel_base = tile * cap
        n_sentinel = n_pad - 1
        pltpu.sync_copy(zeros_hbm.at[pl.ds(0, ACC_ROWS)], hist_v)
        ones = jnp.ones((16,), jnp.float32)

        def blk(b, carry):
            cnt, wr = carry
            base = c * e_half + b * BLK
            pltpu.sync_copy(src_hbm.at[pl.ds(base, BLK)], sblk_v)
            pltpu.sync_copy(dst_hbm.at[pl.ds(base, BLK)], dblk_v)

            def step(i, carry2):
                cnt2, wr2 = carry2
                d16 = dblk_v[pl.ds(i * 16, 16)]
                s16 = sblk_v[pl.ds(i * 16, 16)]
                dl16 = d16 - lo
                mask = (d16 >= lo) & (d16 < lo + RPT)
                dl16c = jnp.where(mask, dl16, RPT)
                plsc.addupdate_scatter(hist_v, [dl16c], ones, mask=mask)
                plsc.store_compressed(psrc_v.at[pl.ds(cnt2, 16)], s16,
                                      mask=mask)
                plsc.store_compressed(pdl_v.at[pl.ds(cnt2, 16)], dl16,
                                      mask=mask)
                cnt2 = cnt2 + jnp.sum(mask.astype(jnp.int32))

                def flush():
                    off = sel_base + wr2 * CH
                    pltpu.sync_copy(psrc_v.at[pl.ds(0, CH)],
                                    ssrc_hbm.at[pl.ds(off, CH)])
                    pltpu.sync_copy(pdl_v.at[pl.ds(0, CH)],
                                    sdl_hbm.at[pl.ds(off, CH)])
                    psrc_v[pl.ds(0, 16)] = psrc_v[pl.ds(CH, 16)]
                    pdl_v[pl.ds(0, 16)] = pdl_v[pl.ds(CH, 16)]
                do = cnt2 >= CH
                lax.cond(do, flush, lambda: None)
                return (jnp.where(do, cnt2 - CH, cnt2),
                        jnp.where(do, wr2 + 1, wr2))

            return lax.fori_loop(0, BLK // 16, step, (cnt, wr))

        cnt, wr = lax.fori_loop(0, n_blk, blk, (jnp.int32(0), jnp.int32(0)))

        spread = (lax.iota(jnp.int32, 16) * 577 + s * 37) & 8191

        def padloop(i, carry):
            keep = lax.iota(jnp.int32, 16) + i * 16 < cnt
            psrc_v[pl.ds(i * 16, 16)] = jnp.where(
                keep, psrc_v[pl.ds(i * 16, 16)], spread + i)
            pdl_v[pl.ds(i * 16, 16)] = jnp.where(
                keep, pdl_v[pl.ds(i * 16, 16)],
                jnp.full((16,), TRASH, jnp.int32))
            return carry
        lax.fori_loop(0, CH // 16, padloop, 0)

        pltpu.sync_copy(psrc_v.at[pl.ds(0, CH)],
                        ssrc_hbm.at[pl.ds(sel_base + wr * CH, CH)])
        pltpu.sync_copy(pdl_v.at[pl.ds(0, CH)],
                        sdl_hbm.at[pl.ds(sel_base + wr * CH, CH)])

        def padloop2(i, carry):
            psrc_v[pl.ds(i * 16, 16)] = spread + i * 3
            pdl_v[pl.ds(i * 16, 16)] = jnp.full((16,), TRASH, jnp.int32)
            return carry
        lax.fori_loop(0, CH // 16, padloop2, 0)
        pltpu.sync_copy(psrc_v.at[pl.ds(0, CH)],
                        ssrc_hbm.at[pl.ds(sel_base + (wr + 1) * CH, CH)])
        pltpu.sync_copy(pdl_v.at[pl.ds(0, CH)],
                        sdl_hbm.at[pl.ds(sel_base + (wr + 1) * CH, CH)])
        nch = wr + jnp.where(cnt > 0, 1, 0)
        nch = nch + (nch & 1)
        psrc_v[pl.ds(0, 16)] = jnp.full((16,), nch, jnp.int32)
        pltpu.sync_copy(psrc_v.at[pl.ds(0, 16)], cnt_hbm.at[pl.ds(tile * 16, 16)])
        pltpu.sync_copy(hist_v.at[pl.ds(0, RPT)],
                        deg_hbm.at[pl.ds(c * n_pad + lo, RPT)])

    return part_kernel


def _make_agg_kernel(n_pad, cap):
    @functools.partial(
        pl.kernel,
        out_type=jax.ShapeDtypeStruct((NC * n_pad, 64), jnp.float32),
        mesh=_sc_mesh(),
        scratch_types=[
            pltpu.VMEM((CH,), jnp.int32),
            pltpu.VMEM((CH,), jnp.int32),
            pltpu.VMEM((CH, 128), jnp.float32),
            pltpu.VMEM((16,), jnp.int32),
            pltpu.VMEM((ACC_ROWS, 64), jnp.float32),
            pltpu.SemaphoreType.DMA,
        ],
        compiler_params=pltpu.CompilerParams(needs_layout_passes=False),
    )
    def agg_kernel(tab_hbm, ssrc_hbm, sdl_hbm, cnt_hbm, zeros_hbm, out_hbm,
                   src_v, dl_v, msg_v, cnt_v, acc_v, sem):
        c = lax.axis_index("c")
        s = lax.axis_index("s")
        tile = c * NS + s
        sel_base = tile * cap
        pltpu.sync_copy(zeros_hbm, acc_v)
        pltpu.sync_copy(cnt_hbm.at[pl.ds(tile * 16, 16)], cnt_v)
        nch = cnt_v[pl.ds(0, 16)][0]
        lane = lax.iota(jnp.int32, 16)

        def chunk(i, carry):
            off = sel_base + i * CH
            pltpu.sync_copy(ssrc_hbm.at[pl.ds(off, CH)], src_v)
            pltpu.sync_copy(sdl_hbm.at[pl.ds(off, CH)], dl_v)
            pltpu.async_copy(tab_hbm.at[src_v], msg_v, sem).wait()

            def group(gi, carry2):
                rows = gi * 16 + lane
                dls = dl_v[pl.ds(gi * 16, 16)]
                for t in range(64):
                    cols = (lane + t) & 63
                    vals = plsc.load_gather(msg_v, [rows, cols])
                    plsc.addupdate_scatter(acc_v, [dls, cols], vals)
                return carry2
            lax.fori_loop(0, CH // 16, group, 0)
            return carry
        lax.fori_loop(0, nch, chunk, 0)
        pltpu.sync_copy(acc_v.at[pl.ds(0, RPT)],
                        out_hbm.at[pl.ds(c * n_pad + s * RPT, RPT)])

    return agg_kernel


def _tc_prologue(x, W1, d0, d1, bn):
    n, d_in = x.shape
    d_h = W1.shape[1]

    def body(x_ref, w_ref, d0_ref, d1_ref, h_ref, g_ref, dv_ref):
        deg = d0_ref[...] + d1_ref[...] + 1.0
        dinv = lax.rsqrt(deg)
        h = jnp.dot(x_ref[...], w_ref[...], preferred_element_type=jnp.float32)
        h_ref[...] = h
        g_ref[:, 0:d_h] = h * dinv
        g_ref[:, d_h:2 * d_h] = jnp.zeros((bn, d_h), jnp.float32)
        dv_ref[...] = dinv

    return pl.pallas_call(
        body,
        grid=(n // bn,),
        in_specs=[
            pl.BlockSpec((bn, d_in), lambda i: (i, 0)),
            pl.BlockSpec((d_in, d_h), lambda i: (0, 0)),
            pl.BlockSpec((bn, 1), lambda i: (i, 0)),
            pl.BlockSpec((bn, 1), lambda i: (i, 0)),
        ],
        out_specs=[
            pl.BlockSpec((bn, d_h), lambda i: (i, 0)),
            pl.BlockSpec((bn, 2 * d_h), lambda i: (i, 0)),
            pl.BlockSpec((bn, 1), lambda i: (i, 0)),
        ],
        out_shape=[
            jax.ShapeDtypeStruct((n, d_h), jnp.float32),
            jax.ShapeDtypeStruct((n, 2 * d_h), jnp.float32),
            jax.ShapeDtypeStruct((n, 1), jnp.float32),
        ],
    )(x, W1, d0, d1)


def _tc_mid(a0, a1, h1, dinv, b1, W2, bn):
    n, d_h = h1.shape

    def body(a0_ref, a1_ref, h1_ref, dv_ref, b_ref, w_ref, h2_ref, g2_ref):
        dv = dv_ref[...]
        z = dv * (a0_ref[...] + a1_ref[...]) + (dv * dv) * h1_ref[...] + b_ref[...]
        h1p = jnp.maximum(z, 0.0)
        h2 = jnp.dot(h1p, w_ref[...], preferred_element_type=jnp.float32)
        h2_ref[...] = h2
        g2_ref[:, 0:d_h] = h2 * dv
        g2_ref[:, d_h:2 * d_h] = jnp.zeros((bn, d_h), jnp.float32)

    return pl.pallas_call(
        body,
        grid=(n // bn,),
        in_specs=[
            pl.BlockSpec((bn, d_h), lambda i: (i, 0)),
            pl.BlockSpec((bn, d_h), lambda i: (i, 0)),
            pl.BlockSpec((bn, d_h), lambda i: (i, 0)),
            pl.BlockSpec((bn, 1), lambda i: (i, 0)),
            pl.BlockSpec((1, d_h), lambda i: (0, 0)),
            pl.BlockSpec((d_h, d_h), lambda i: (0, 0)),
        ],
        out_specs=[
            pl.BlockSpec((bn, d_h), lambda i: (i, 0)),
            pl.BlockSpec((bn, 2 * d_h), lambda i: (i, 0)),
        ],
        out_shape=[
            jax.ShapeDtypeStruct((n, d_h), jnp.float32),
            jax.ShapeDtypeStruct((n, 2 * d_h), jnp.float32),
        ],
    )(a0, a1, h1, dinv, b1, W2)


def _tc_final(a0, a1, h2, dinv, b2, batch_row, Wfc, bfc):
    n, d_h = h2.shape
    ncls = Wfc.shape[1]
    g = NUM_GRAPHS

    def body(a0_ref, a1_ref, h2_ref, dv_ref, b_ref, bt_ref, wfc_ref, bfc_ref,
             o_ref):
        dv = dv_ref[...]
        z = dv * (a0_ref[...] + a1_ref[...]) + (dv * dv) * h2_ref[...] + b_ref[...]
        hp = jnp.maximum(z, 0.0)
        gids = lax.broadcasted_iota(jnp.int32, (g, n), 0)
        m_t = (bt_ref[...] == gids).astype(jnp.float32)
        sums = jnp.dot(m_t, hp, preferred_element_type=jnp.float32)
        cnt = jnp.dot(m_t, jnp.ones((n, 1), jnp.float32),
                      preferred_element_type=jnp.float32)
        pooled = sums / jnp.maximum(cnt, 1.0)
        o_ref[...] = (jnp.dot(pooled, wfc_ref[...],
                              preferred_element_type=jnp.float32)
                      + bfc_ref[...])

    return pl.pallas_call(
        body,
        out_shape=jax.ShapeDtypeStruct((g, ncls), jnp.float32),
    )(a0, a1, h2, dinv, b2, batch_row, Wfc, bfc)


def kernel(x, edge_index, batch, W1, b1, W2, b2, Wfc, bfc):
    n, d_in = x.shape
    e = edge_index.shape[1]
    d_h = W1.shape[1]

    n_pad = NS * RPT
    e_half = -(-e // (NC * BLK)) * BLK
    e_pad = NC * e_half
    cap = e_half + 2 * CH
    bn = n_pad // 4
    while bn % 8 != 0 or n_pad % bn != 0:
        bn //= 2

    epad = e_pad - e
    src = jnp.concatenate(
        [edge_index[0], jnp.arange(epad, dtype=jnp.int32) & 8191])
    dst = jnp.concatenate([edge_index[1], jnp.full((epad,), n, jnp.int32)])
    x_p = jnp.concatenate([x, jnp.zeros((n_pad - n, d_in), jnp.float32)])
    batch_row = jnp.concatenate(
        [batch, jnp.full((n_pad - n,), NUM_GRAPHS, jnp.int32)]).reshape(1, n_pad)
    zeros_hist = jnp.zeros((ACC_ROWS,), jnp.float32)
    zeros_acc = jnp.zeros((ACC_ROWS, 64), jnp.float32)

    ssrc, sdl, cnts, deg = _make_partition_kernel(e_half, n_pad, cap)(
        src, dst, zeros_hist)
    deg2 = deg.reshape(NC, n_pad)
    d0 = deg2[0].reshape(n_pad, 1)
    d1 = deg2[1].reshape(n_pad, 1)

    h1, g1, dinv = _tc_prologue(x_p, W1, d0, d1, bn)

    agg_call = _make_agg_kernel(n_pad, cap)
    agg1 = agg_call(g1, ssrc, sdl, cnts, zeros_acc).reshape(NC, n_pad, d_h)
    h2, g2 = _tc_mid(agg1[0], agg1[1], h1, dinv, b1.reshape(1, -1), W2, bn)
    agg2 = agg_call(g2, ssrc, sdl, cnts, zeros_acc).reshape(NC, n_pad, d_h)

    return _tc_final(agg2[0], agg2[1], h2, dinv, b2.reshape(1, -1), batch_row,
                     Wfc, bfc.reshape(1, -1))
